# Initial kernel scaffold; baseline (speedup 1.0000x reference)
import jax
import jax.numpy as jnp
from jax.experimental import pallas as pl


def kernel(logit, target):
    raise NotImplementedError("write your pallas kernel here")


# speedup vs baseline: 1.7601x; 1.7601x over previous
import jax
import jax.numpy as jnp
from jax.experimental import pallas as pl
from jax.experimental.pallas import tpu as pltpu

N_ROWS = 1024
N_CLASSES = 100000
IGNORE = 0
SMOOTH = 0.1
FILL = SMOOTH / (N_CLASSES - 1)
CONF = 1.0 - SMOOTH

BLK = 2048
GRID = (N_CLASSES + BLK - 1) // BLK


def _loss_body(logit_ref, tgt_ref, out_ref):
    j = pl.program_id(0)
    base = j * BLK
    x = logit_ref[...]
    t = tgt_ref[...]
    col = jax.lax.broadcasted_iota(jnp.int32, x.shape, 1) + base
    x = jnp.where(col < N_CLASSES, x, 0.0)
    w = jnp.where(col == t, CONF, FILL)
    w = jnp.where(t == IGNORE, 0.0, w)
    s = jnp.sum(x * w)

    @pl.when(j == 0)
    def _():
        out_ref[0, 0] = 0.0

    out_ref[0, 0] += s


def kernel(logit, target):
    t2 = target.astype(jnp.int32).reshape(N_ROWS, 1)
    res = pl.pallas_call(
        _loss_body,
        grid=(GRID,),
        in_specs=[
            pl.BlockSpec((N_ROWS, BLK), lambda j: (0, j)),
            pl.BlockSpec((N_ROWS, 1), lambda j: (0, 0)),
        ],
        out_specs=pl.BlockSpec(memory_space=pltpu.SMEM),
        out_shape=jax.ShapeDtypeStruct((1, 1), jnp.float32),
    )(logit, t2)
    return -res[0, 0]
